# baseline (speedup 1.0000x reference)
import jax
import jax.numpy as jnp
from jax.experimental import pallas as pl

_IDX = 1
_BM = 512


def _double_kernel(a_ref, o_ref):
    o_ref[...] = a_ref[...] + a_ref[...]


def kernel(a):
    n, m, k = a.shape
    grid = (m // _BM,)
    out = pl.pallas_call(
        _double_kernel,
        grid=grid,
        in_specs=[pl.BlockSpec((1, _BM, k), lambda i: (_IDX, i, 0))],
        out_specs=pl.BlockSpec((1, _BM, k), lambda i: (0, i, 0)),
        out_shape=jax.ShapeDtypeStruct((1, m, k), a.dtype),
    )(a)
    return out.reshape(m, k)


# speedup vs baseline: 1.0300x; 1.0300x over previous
import jax
import jax.numpy as jnp
from jax.experimental import pallas as pl

_IDX = 1
_BM = 1024


def _double_kernel(a_ref, o_ref):
    o_ref[...] = a_ref[...] + a_ref[...]


def kernel(a):
    n, m, k = a.shape
    a2 = a.reshape(n * m, k)
    nblk = m // _BM
    out = pl.pallas_call(
        _double_kernel,
        grid=(nblk,),
        in_specs=[pl.BlockSpec((_BM, k), lambda i: (_IDX * (m // _BM) + i, 0))],
        out_specs=pl.BlockSpec((_BM, k), lambda i: (i, 0)),
        out_shape=jax.ShapeDtypeStruct((m, k), a.dtype),
    )(a2)
    return out
